# Initial kernel scaffold; baseline (speedup 1.0000x reference)
import jax
import jax.numpy as jnp
from jax.experimental import pallas as pl


def kernel(X, table):
    raise NotImplementedError("write your pallas kernel here")


# speedup vs baseline: 4.8079x; 4.8079x over previous
import functools

import jax
import jax.numpy as jnp
from jax import lax
from jax.experimental import pallas as pl
from jax.experimental.pallas import tpu as pltpu
from jax.experimental.pallas import tpu_sc as plsc

EMBED_DIM = 32
_NC = 2
_NS = 16
_NW = _NC * _NS
_CHUNK = 1024


@functools.lru_cache(maxsize=None)
def _make_gather(n_rows: int):
    per_w = n_rows // _NW
    assert per_w * _NW == n_rows
    n_chunks = per_w // _CHUNK
    assert n_chunks * _CHUNK == per_w
    mesh = plsc.VectorSubcoreMesh(core_axis_name="c", subcore_axis_name="s")

    @functools.partial(
        pl.kernel,
        mesh=mesh,
        out_type=jax.ShapeDtypeStruct((n_rows, EMBED_DIM), jnp.float32),
        scratch_types=[
            pltpu.VMEM((_CHUNK,), jnp.int32),
            pltpu.VMEM((_CHUNK, EMBED_DIM), jnp.float32),
            pltpu.SemaphoreType.DMA,
        ],
        compiler_params=pltpu.CompilerParams(use_tc_tiling_on_sc=False),
    )
    def gather(idx_hbm, table_hbm, out_hbm, idx_v, rows_v, sem):
        wid = lax.axis_index("s") * _NC + lax.axis_index("c")
        base = wid * per_w

        def chunk_body(i, carry):
            off = base + i * _CHUNK
            pltpu.sync_copy(idx_hbm.at[pl.ds(off, _CHUNK)], idx_v)
            pltpu.async_copy(table_hbm.at[idx_v], rows_v, sem).wait()
            pltpu.sync_copy(rows_v, out_hbm.at[pl.ds(off, _CHUNK)])
            return carry

        lax.fori_loop(0, n_chunks, chunk_body, 0)

    return gather


def kernel(X, table):
    flat = X.reshape(-1).astype(jnp.int32)
    out = _make_gather(flat.shape[0])(flat, table)
    return out.reshape(X.shape + (EMBED_DIM,))


# speedup vs baseline: 4.9561x; 1.0308x over previous
import functools

import jax
import jax.numpy as jnp
from jax import lax
from jax.experimental import pallas as pl
from jax.experimental.pallas import tpu as pltpu
from jax.experimental.pallas import tpu_sc as plsc

EMBED_DIM = 32
_NC = 2
_NS = 16
_NW = _NC * _NS
_CHUNK = 1024


@functools.lru_cache(maxsize=None)
def _make_gather(n_rows: int):
    per_w = n_rows // _NW
    assert per_w * _NW == n_rows
    n_outer = per_w // (2 * _CHUNK)
    assert n_outer * 2 * _CHUNK == per_w
    mesh = plsc.VectorSubcoreMesh(core_axis_name="c", subcore_axis_name="s")

    @functools.partial(
        pl.kernel,
        mesh=mesh,
        out_type=jax.ShapeDtypeStruct((n_rows, EMBED_DIM), jnp.float32),
        scratch_types=[
            pltpu.VMEM((_CHUNK,), jnp.int32),
            pltpu.VMEM((_CHUNK,), jnp.int32),
            pltpu.VMEM((_CHUNK, EMBED_DIM), jnp.float32),
            pltpu.VMEM((_CHUNK, EMBED_DIM), jnp.float32),
            pltpu.SemaphoreType.DMA,
            pltpu.SemaphoreType.DMA,
            pltpu.SemaphoreType.DMA,
        ],
        compiler_params=pltpu.CompilerParams(use_tc_tiling_on_sc=False),
    )
    def gather(idx_hbm, table_hbm, out_hbm, idx0, idx1, rows0, rows1,
               sem_g, sem_w0, sem_w1):
        wid = lax.axis_index("s") * _NC + lax.axis_index("c")
        base = wid * per_w
        bufs = ((idx0, rows0, sem_w0), (idx1, rows1, sem_w1))

        def outer(g, carry):
            for b in range(2):
                idx_v, rows_v, sem_w = bufs[b]
                off = base + (g * 2 + b) * _CHUNK

                @pl.when(g > 0)
                def _():
                    pltpu.make_async_copy(
                        rows_v, out_hbm.at[pl.ds(0, _CHUNK)], sem_w).wait()

                pltpu.sync_copy(idx_hbm.at[pl.ds(off, _CHUNK)], idx_v)
                pltpu.async_copy(table_hbm.at[idx_v], rows_v, sem_g).wait()
                pltpu.async_copy(rows_v, out_hbm.at[pl.ds(off, _CHUNK)], sem_w)
            return carry

        lax.fori_loop(0, n_outer, outer, 0)
        pltpu.make_async_copy(rows0, out_hbm.at[pl.ds(0, _CHUNK)], sem_w0).wait()
        pltpu.make_async_copy(rows1, out_hbm.at[pl.ds(0, _CHUNK)], sem_w1).wait()

    return gather


def kernel(X, table):
    flat = X.reshape(-1).astype(jnp.int32)
    out = _make_gather(flat.shape[0])(flat, table)
    return out.reshape(X.shape + (EMBED_DIM,))


# speedup vs baseline: 5.0296x; 1.0148x over previous
import functools

import jax
import jax.numpy as jnp
from jax import lax
from jax.experimental import pallas as pl
from jax.experimental.pallas import tpu as pltpu
from jax.experimental.pallas import tpu_sc as plsc

EMBED_DIM = 32
_NC = 2
_NS = 16
_NW = _NC * _NS
_CHUNK = 512
_NBUF = 4


@functools.lru_cache(maxsize=None)
def _make_gather(n_rows: int):
    per_w = n_rows // _NW
    assert per_w * _NW == n_rows
    n_outer = per_w // (_NBUF * _CHUNK)
    assert n_outer * _NBUF * _CHUNK == per_w
    mesh = plsc.VectorSubcoreMesh(core_axis_name="c", subcore_axis_name="s")

    @functools.partial(
        pl.kernel,
        mesh=mesh,
        out_type=jax.ShapeDtypeStruct((n_rows, EMBED_DIM), jnp.float32),
        scratch_types=(
            [pltpu.VMEM((_CHUNK,), jnp.int32) for _ in range(_NBUF)]
            + [pltpu.VMEM((_CHUNK, EMBED_DIM), jnp.float32) for _ in range(_NBUF)]
            + [pltpu.SemaphoreType.DMA for _ in range(2 * _NBUF)]
        ),
        compiler_params=pltpu.CompilerParams(use_tc_tiling_on_sc=False),
    )
    def gather(idx_hbm, table_hbm, out_hbm, *scratch):
        idxs = scratch[:_NBUF]
        rows = scratch[_NBUF:2 * _NBUF]
        sem_g = scratch[2 * _NBUF:3 * _NBUF]
        sem_w = scratch[3 * _NBUF:]
        wid = lax.axis_index("s") * _NC + lax.axis_index("c")
        base = wid * per_w

        def start_gather(b, chunk_id):
            off = base + chunk_id * _CHUNK
            pltpu.sync_copy(idx_hbm.at[pl.ds(off, _CHUNK)], idxs[b])
            pltpu.async_copy(table_hbm.at[idxs[b]], rows[b], sem_g[b])

        for b in range(_NBUF):
            start_gather(b, b)

        def outer(g, carry):
            for b in range(_NBUF):
                off = base + (g * _NBUF + b) * _CHUNK
                pltpu.make_async_copy(
                    table_hbm.at[idxs[b]], rows[b], sem_g[b]).wait()
                pltpu.async_copy(rows[b], out_hbm.at[pl.ds(off, _CHUNK)],
                                 sem_w[b])
            for b in range(_NBUF):
                @pl.when(g < n_outer - 1)
                def _():
                    pltpu.make_async_copy(
                        rows[b], out_hbm.at[pl.ds(0, _CHUNK)], sem_w[b]).wait()
                    start_gather(b, (g + 1) * _NBUF + b)
            return carry

        lax.fori_loop(0, n_outer, outer, 0)
        for b in range(_NBUF):
            pltpu.make_async_copy(
                rows[b], out_hbm.at[pl.ds(0, _CHUNK)], sem_w[b]).wait()

    return gather


def kernel(X, table):
    flat = X.reshape(-1).astype(jnp.int32)
    out = _make_gather(flat.shape[0])(flat, table)
    return out.reshape(X.shape + (EMBED_DIM,))
